# baseline (speedup 1.0000x reference)
import functools

import jax
import jax.numpy as jnp
from jax import lax
from jax.experimental import pallas as pl
from jax.experimental.pallas import tpu as pltpu
from jax.experimental.pallas import tpu_sc as plsc

E = 8
D = 768
F = 2048
T = 2048
P = 2 * T
TM = 256
NT = 24
NPAD = NT * TM
NS = 16


def _router_body(x_ref, wr_ref, e1_ref, e2_ref, w1_ref, w2_ref, cnt_ref):
    logits = lax.dot_general(
        x_ref[...], wr_ref[...], (((1,), (1,)), ((), ())),
        preferred_element_type=jnp.float32)
    idx = lax.broadcasted_iota(jnp.int32, logits.shape, 1)
    m1 = jnp.max(logits, axis=1, keepdims=True)
    i1 = jnp.min(jnp.where(logits == m1, idx, E), axis=1, keepdims=True)
    masked = jnp.where(idx == i1, -jnp.inf, logits)
    m2 = jnp.max(masked, axis=1, keepdims=True)
    i2 = jnp.min(jnp.where(masked == m2, idx, E), axis=1, keepdims=True)
    w1 = 1.0 / (1.0 + jnp.exp(m2 - m1))
    e1_ref[...] = i1
    e2_ref[...] = i2
    w1_ref[...] = w1
    w2_ref[...] = 1.0 - w1
    tb = lax.broadcasted_iota(jnp.int32, (T, NS), 1)
    tokb = lax.broadcasted_iota(jnp.int32, (T, NS), 0) // 128
    bmask = (tb == tokb).astype(jnp.float32)
    m1f = (idx == i1).astype(jnp.float32)
    m2f = (idx == i2).astype(jnp.float32)
    h1 = lax.dot_general(bmask, m1f, (((0,), (0,)), ((), ())),
                         preferred_element_type=jnp.float32)
    h2 = lax.dot_general(bmask, m2f, (((0,), (0,)), ((), ())),
                         preferred_element_type=jnp.float32)
    h1p = jnp.pad(h1, ((0, 0), (0, 16 - E)))
    h2p = jnp.pad(h2, ((0, 0), (0, 16 - E)))
    cnt_ref[...] = jnp.concatenate([h1p, h2p], axis=0).astype(jnp.int32)


def _sort_scatter_body(ef_hbm, cnt_hbm, x_hbm, xs_hbm, pos_hbm, te_hbm,
                       ef_v, allcnt_v, ptr_v, dest_v, te_v, rows_v, sem):
    cid = lax.axis_index("c")
    sid = lax.axis_index("s")
    wid = sid * 2 + cid
    lane = lax.iota(jnp.int32, 16)

    t0 = (wid & (NS - 1)) * 128
    drows = pltpu.async_copy(x_hbm.at[pl.ds(t0, 128)], rows_v, sem)
    pltpu.sync_copy(ef_hbm.at[wid], ef_v)
    pltpu.sync_copy(cnt_hbm, allcnt_v)
    total = jnp.zeros((16,), jnp.int32)
    pref = jnp.zeros((16,), jnp.int32)
    for w in range(2 * NS):
        row = allcnt_v[w, :]
        total = total + row
        pref = pref + jnp.where(w < wid, row, 0)
    padded = ((total + (TM - 1)) // TM) * TM
    incl = plsc.cumsum(padded)
    base = incl - padded
    ptr = base + pref

    for j in range(8):
        v = ef_v[pl.ds(j * 16, 16)]
        ptr_v[...] = ptr
        myp = plsc.load_gather(ptr_v, [v])
        dest = jnp.zeros((16,), jnp.int32)
        for e in range(E):
            m = v == e
            r = plsc.cumsum(jnp.where(m, 1, 0))
            dest = jnp.where(m, myp + r - 1, dest)
            ptr = ptr + jnp.where(lane == e, r[15], 0)
        dest_v[0, pl.ds(j * 16, 16)] = dest

    pltpu.sync_copy(dest_v.at[0], pos_hbm.at[pl.ds(wid * 128, 128)])
    drows.wait()
    pltpu.async_copy(rows_v, xs_hbm.at[dest_v.at[0]], sem).wait()

    @pl.when((cid == 0) & (sid == 0))
    def _te():
        for h in range(2):
            row0 = (lane + h * 16) * TM
            te = jnp.full((16,), 2 * E - 1, jnp.int32)
            for e in range(E):
                be = jnp.sum(jnp.where(lane == e, base, 0))
                pe = jnp.sum(jnp.where(lane == e, padded, 0))
                m = (row0 >= be) & (row0 < be + pe)
                te = jnp.where(m, e, te)
            te_v[pl.ds(h * 16, 16)] = te
        pltpu.sync_copy(te_v, te_hbm)


def _expert_body(te_ref, xs_ref, gu_ref, dn_ref, yw_ref, gub_s, dnb_s):
    j = pl.program_id(0)
    te = te_ref[j]

    @pl.when(te < E)
    def _():
        changed = jnp.logical_or(j == 0, te_ref[jnp.maximum(j - 1, 0)] != te)

        @pl.when(changed)
        def _cvt():
            gub_s[...] = gu_ref[0].astype(jnp.bfloat16)
            dnb_s[...] = dn_ref[0].astype(jnp.bfloat16)

        xb = xs_ref[...].astype(jnp.bfloat16)
        FC = F // 2
        acc = None
        for c in range(2):
            g = jnp.dot(xb, gub_s[:, c * FC:(c + 1) * FC],
                        preferred_element_type=jnp.float32)
            u = jnp.dot(xb, gub_s[:, F + c * FC:F + (c + 1) * FC],
                        preferred_element_type=jnp.float32)
            a = (g * jax.nn.sigmoid(g) * u).astype(jnp.bfloat16)
            yc = jnp.dot(a, dnb_s[c * FC:(c + 1) * FC, :],
                         preferred_element_type=jnp.float32)
            acc = yc if acc is None else acc + yc
        yw_ref[...] = acc


def _combine_body(yw_hbm, pos_hbm, w1_hbm, w2_hbm, out_hbm,
                  p0_v, p1_v, wa_v, wb_v, b0a, b1a, b0b, b1b,
                  sem0, sem1, sem2, sem3):
    cid = lax.axis_index("c")
    sid = lax.axis_index("s")
    wid = sid * 2 + cid
    pltpu.sync_copy(pos_hbm.at[pl.ds(wid * 64, 64)], p0_v)
    pltpu.sync_copy(pos_hbm.at[pl.ds(T + wid * 64, 64)], p1_v)
    pltpu.sync_copy(w1_hbm.at[pl.ds(wid * 64, 64)], wa_v.at[pl.ds(0, 64)])
    pltpu.sync_copy(w2_hbm.at[pl.ds(wid * 64, 64)], wb_v.at[pl.ds(0, 64)])
    dA0 = pltpu.async_copy(yw_hbm.at[p0_v.at[pl.ds(0, 32)]], b0a, sem0)
    dA1 = pltpu.async_copy(yw_hbm.at[p1_v.at[pl.ds(0, 32)]], b1a, sem1)
    dB0 = pltpu.async_copy(yw_hbm.at[p0_v.at[pl.ds(32, 32)]], b0b, sem2)
    dB1 = pltpu.async_copy(yw_hbm.at[p1_v.at[pl.ds(32, 32)]], b1b, sem3)

    for c, (d0, d1, b0, b1) in enumerate(((dA0, dA1, b0a, b1a),
                                          (dB0, dB1, b0b, b1b))):
        d0.wait()
        d1.wait()

        def body(i, carry):
            wa = wa_v[pl.ds(c * 32 + i, 16)][0]
            wb = wb_v[pl.ds(c * 32 + i, 16)][0]
            for q in range(D // 16):
                b0[i, pl.ds(q * 16, 16)] = (b0[i, pl.ds(q * 16, 16)] * wa
                                            + b1[i, pl.ds(q * 16, 16)] * wb)
            return carry

        lax.fori_loop(0, 32, body, 0)
        pltpu.sync_copy(b0, out_hbm.at[pl.ds(wid * 64 + c * 32, 32)])


def kernel(hidden_states, router_weight, gate_up_proj, down_proj):
    B, S, _ = hidden_states.shape
    x32 = hidden_states.reshape(B * S, D)

    e1, e2, w1, w2, cnt16 = pl.pallas_call(
        _router_body,
        in_specs=[pl.BlockSpec((T, D), lambda: (0, 0)),
                  pl.BlockSpec((E, D), lambda: (0, 0))],
        out_specs=[pl.BlockSpec((T, 1), lambda: (0, 0))] * 4
        + [pl.BlockSpec((2 * NS, 16), lambda: (0, 0))],
        out_shape=[jax.ShapeDtypeStruct((T, 1), jnp.int32),
                   jax.ShapeDtypeStruct((T, 1), jnp.int32),
                   jax.ShapeDtypeStruct((T, 1), jnp.float32),
                   jax.ShapeDtypeStruct((T, 1), jnp.float32),
                   jax.ShapeDtypeStruct((2 * NS, 16), jnp.int32)],
    )(x32, router_weight)

    ef = jnp.concatenate([e1, e2], axis=0).reshape(32, 128)

    mesh = plsc.VectorSubcoreMesh(core_axis_name="c", subcore_axis_name="s",
                                  num_cores=2, num_subcores=NS)
    sc_params = pltpu.CompilerParams(needs_layout_passes=False)

    sort_scatter = functools.partial(
        pl.kernel,
        out_type=[jax.ShapeDtypeStruct((NPAD, D), jnp.float32),
                  jax.ShapeDtypeStruct((P,), jnp.int32),
                  jax.ShapeDtypeStruct((32,), jnp.int32)],
        mesh=mesh,
        scratch_types=[
            pltpu.VMEM((128,), jnp.int32),
            pltpu.VMEM((2 * NS, 16), jnp.int32),
            pltpu.VMEM((16,), jnp.int32),
            pltpu.VMEM((2, 128), jnp.int32),
            pltpu.VMEM((32,), jnp.int32),
            pltpu.VMEM((128, D), jnp.float32),
            pltpu.SemaphoreType.DMA,
        ],
        compiler_params=sc_params,
    )(_sort_scatter_body)
    xs, pos, te = sort_scatter(ef, cnt16, x32)

    grid_spec = pltpu.PrefetchScalarGridSpec(
        num_scalar_prefetch=1,
        grid=(NT,),
        in_specs=[
            pl.BlockSpec((TM, D), lambda j, te_r: (j, 0)),
            pl.BlockSpec((1, D, 2 * F), lambda j, te_r: (te_r[j] & 7, 0, 0)),
            pl.BlockSpec((1, F, D), lambda j, te_r: (te_r[j] & 7, 0, 0)),
        ],
        out_specs=pl.BlockSpec((TM, D), lambda j, te_r: (j, 0)),
        scratch_shapes=[pltpu.VMEM((D, 2 * F), jnp.bfloat16),
                        pltpu.VMEM((F, D), jnp.bfloat16)],
    )
    yw = pl.pallas_call(
        _expert_body,
        grid_spec=grid_spec,
        out_shape=jax.ShapeDtypeStruct((NPAD, D), jnp.float32),
        compiler_params=pltpu.CompilerParams(
            dimension_semantics=("arbitrary",)),
    )(te, xs, gate_up_proj, down_proj)

    combine = functools.partial(
        pl.kernel,
        out_type=jax.ShapeDtypeStruct((T, D), jnp.float32),
        mesh=mesh,
        scratch_types=[
            pltpu.VMEM((64,), jnp.int32),
            pltpu.VMEM((64,), jnp.int32),
            pltpu.VMEM((80,), jnp.float32),
            pltpu.VMEM((80,), jnp.float32),
            pltpu.VMEM((32, D), jnp.float32),
            pltpu.VMEM((32, D), jnp.float32),
            pltpu.VMEM((32, D), jnp.float32),
            pltpu.VMEM((32, D), jnp.float32),
            pltpu.SemaphoreType.DMA,
            pltpu.SemaphoreType.DMA,
            pltpu.SemaphoreType.DMA,
            pltpu.SemaphoreType.DMA,
        ],
        compiler_params=sc_params,
    )(_combine_body)
    out = combine(yw, pos, w1.reshape(T), w2.reshape(T))
    return out.reshape(B, S, D)
